# Initial kernel scaffold; baseline (speedup 1.0000x reference)
import jax
import jax.numpy as jnp
from jax.experimental import pallas as pl


def kernel(x):
    raise NotImplementedError("write your pallas kernel here")


# speedup vs baseline: 2.0648x; 2.0648x over previous
import jax
import jax.numpy as jnp
from jax.experimental import pallas as pl


def _copy_kernel(x_ref, o_ref):
    o_ref[...] = x_ref[...]


def kernel(x):
    n, r, c = x.shape
    BR = 256
    grid = (n, r // BR)
    return pl.pallas_call(
        _copy_kernel,
        grid=grid,
        in_specs=[pl.BlockSpec((1, BR, c), lambda i, j: (n - 1 - i, j, 0))],
        out_specs=pl.BlockSpec((1, BR, c), lambda i, j: (i, j, 0)),
        out_shape=jax.ShapeDtypeStruct((n, r, c), x.dtype),
    )(x)
